# Initial kernel scaffold; baseline (speedup 1.0000x reference)
import jax
import jax.numpy as jnp
from jax.experimental import pallas as pl


def kernel(variables, factors, edge_index, edge_attr, batch_idx, v2f_msg_W, v2f_msg_b, v2f_comb_W, v2f_comb_b, f2v_msg_W, f2v_msg_b, f2v_comb_W, f2v_comb_b):
    raise NotImplementedError("write your pallas kernel here")


# speedup vs baseline: 2.2481x; 2.2481x over previous
import functools

import jax
import jax.numpy as jnp
from jax import lax
from jax.experimental import pallas as pl
from jax.experimental.pallas import tpu as pltpu
from jax.experimental.pallas import tpu_sc as plsc

D = 128
H = D // 2
L = 16
NS = 16
ROWS_PAD = 10240
RPT = ROWS_PAD // NS
CH = 128


def _cdiv(a, b):
    return (a + b - 1) // b


def _proj_body(f_ref, v_ref, w1a_ref, w1b_ref, w2a_ref, w2b_ref,
               b1_ref, b2_ref,
               pf1_lo, pf1_hi, pv1_lo, pv1_hi,
               pv2_lo, pv2_hi, pf2_lo, pf2_hi):
    f = f_ref[...]
    v = v_ref[...]
    pf1 = jnp.dot(f, w1a_ref[...], preferred_element_type=jnp.float32) + b1_ref[...]
    pv1 = jnp.dot(v, w1b_ref[...], preferred_element_type=jnp.float32)
    pv2 = jnp.dot(v, w2a_ref[...], preferred_element_type=jnp.float32) + b2_ref[...]
    pf2 = jnp.dot(f, w2b_ref[...], preferred_element_type=jnp.float32)
    pf1_lo[...], pf1_hi[...] = pf1[:, :H], pf1[:, H:]
    pv1_lo[...], pv1_hi[...] = pv1[:, :H], pv1[:, H:]
    pv2_lo[...], pv2_hi[...] = pv2[:, :H], pv2[:, H:]
    pf2_lo[...], pf2_hi[...] = pf2[:, :H], pf2[:, H:]


def _projections(factors_p, variables_p, w1a, w1b, w2a, w2b, b1, b2):
    n = factors_p.shape[0]
    blk = 2048
    grid = (_cdiv(n, blk),)
    row_spec = pl.BlockSpec((blk, D), lambda i: (i, 0))
    half_spec = pl.BlockSpec((blk, H), lambda i: (i, 0))
    w_spec = pl.BlockSpec((D, D), lambda i: (0, 0))
    b_spec = pl.BlockSpec((1, D), lambda i: (0, 0))
    out = jax.ShapeDtypeStruct((n, H), jnp.float32)
    return pl.pallas_call(
        _proj_body,
        grid=grid,
        in_specs=[row_spec, row_spec, w_spec, w_spec, w_spec, w_spec,
                  b_spec, b_spec],
        out_specs=[half_spec] * 8,
        out_shape=[out] * 8,
    )(factors_p, variables_p, w1a, w1b, w2a, w2b, b1, b2)


def _comb_body(f_ref, afl_ref, afh_ref, v_ref, avl_ref, avh_ref,
               wc1a_ref, wc1bl_ref, wc1bh_ref,
               wc2a_ref, wc2bl_ref, wc2bh_ref,
               b3_ref, b4_ref, outf_ref, outv_ref):
    f = f_ref[...]
    v = v_ref[...]
    nf = (jnp.dot(f, wc1a_ref[...], preferred_element_type=jnp.float32)
          + jnp.dot(afl_ref[...], wc1bl_ref[...], preferred_element_type=jnp.float32)
          + jnp.dot(afh_ref[...], wc1bh_ref[...], preferred_element_type=jnp.float32)
          + b3_ref[...])
    outf_ref[...] = jnp.maximum(nf, 0.0)
    nv = (jnp.dot(v, wc2a_ref[...], preferred_element_type=jnp.float32)
          + jnp.dot(avl_ref[...], wc2bl_ref[...], preferred_element_type=jnp.float32)
          + jnp.dot(avh_ref[...], wc2bh_ref[...], preferred_element_type=jnp.float32)
          + b4_ref[...])
    outv_ref[...] = v + jnp.maximum(nv, 0.0)


def _combine(factors, af_lo, af_hi, variables, av_lo, av_hi,
             wc1a, wc1b_lo, wc1b_hi, wc2a, wc2b_lo, wc2b_hi, b3, b4):
    n = factors.shape[0]
    blk = 2000
    grid = (_cdiv(n, blk),)
    row_spec = pl.BlockSpec((blk, D), lambda i: (i, 0))
    half_spec = pl.BlockSpec((blk, H), lambda i: (i, 0))
    w_spec = pl.BlockSpec((D, D), lambda i: (0, 0))
    wh_spec = pl.BlockSpec((H, D), lambda i: (0, 0))
    b_spec = pl.BlockSpec((1, D), lambda i: (0, 0))
    out = jax.ShapeDtypeStruct((n, D), jnp.float32)
    return pl.pallas_call(
        _comb_body,
        grid=grid,
        in_specs=[row_spec, half_spec, half_spec, row_spec, half_spec, half_spec,
                  w_spec, wh_spec, wh_spec, w_spec, wh_spec, wh_spec,
                  b_spec, b_spec],
        out_specs=[row_spec, row_spec],
        out_shape=[out, out],
    )(factors, af_lo, af_hi, variables, av_lo, av_hi,
      wc1a, wc1b_lo, wc1b_hi, wc2a, wc2b_lo, wc2b_hi, b3, b4)


def _sc_edge_aggregate(tables, dst_p, src_p, ept):
    pf1_lo, pf1_hi, pv1_lo, pv1_hi, pv2_lo, pv2_hi, pf2_lo, pf2_hi = tables
    nchunk = ept // CH
    mesh = plsc.VectorSubcoreMesh(core_axis_name="c", subcore_axis_name="s")
    half = jax.ShapeDtypeStruct((ROWS_PAD, H), jnp.float32)

    @functools.partial(
        pl.kernel,
        mesh=mesh,
        compiler_params=pltpu.CompilerParams(use_tc_tiling_on_sc=False),
        out_type=[half, half, half, half],
        scratch_types=[
            pltpu.VMEM((2, CH, H), jnp.float32),
            pltpu.VMEM((2, CH, H), jnp.float32),
            pltpu.VMEM((2, CH), jnp.int32),
            pltpu.VMEM((2, CH), jnp.int32),
            pltpu.VMEM_SHARED((ROWS_PAD, H), jnp.float32),
            pltpu.SemaphoreType.DMA,
            pltpu.SemaphoreType.DMA,
        ],
    )
    def edge_kernel(pf1l_hbm, pf1h_hbm, pv1l_hbm, pv1h_hbm,
                    pv2l_hbm, pv2h_hbm, pf2l_hbm, pf2h_hbm,
                    dst_hbm, src_hbm,
                    outfl_hbm, outfh_hbm, outvl_hbm, outvh_hbm,
                    buf_a, buf_b, idx1, idx2, aggr, sem_a, sem_b):
        c = lax.axis_index("c")
        s = lax.axis_index("s")
        row0 = s * RPT

        def zero_aggr():
            @pl.loop(0, CH)
            def _(r):
                for j in range(H // L):
                    buf_a[0, r, pl.ds(j * L, L)] = jnp.zeros((L,), jnp.float32)

            for k in range(RPT // CH):
                pltpu.sync_copy(buf_a.at[0], aggr.at[pl.ds(row0 + k * CH, CH)])

        def run_direction(t1_hbm, i1_hbm, t2_hbm, i2_hbm, out_hbm):
            zero_aggr()
            plsc.subcore_barrier()

            @pl.loop(0, nchunk)
            def _(g):
                base = s * ept + g * CH
                pltpu.sync_copy(i1_hbm.at[pl.ds(base, CH)], idx1.at[0])
                pltpu.sync_copy(i2_hbm.at[pl.ds(base, CH)], idx2.at[0])
                pltpu.async_copy(t1_hbm.at[idx1.at[0]], buf_a.at[0], sem_a).wait()
                pltpu.async_copy(t2_hbm.at[idx2.at[0]], buf_b.at[0], sem_b).wait()

                @pl.loop(0, CH)
                def _(r):
                    for j in range(H // L):
                        sl = pl.ds(j * L, L)
                        m = buf_a[0, r, sl] + buf_b[0, r, sl]
                        buf_a[0, r, sl] = jnp.maximum(m, 0.0)

                pltpu.sync_copy(buf_a.at[0], aggr.at[idx1.at[0]], add=True)

            plsc.subcore_barrier()
            pltpu.sync_copy(aggr.at[pl.ds(row0, RPT)], out_hbm.at[pl.ds(row0, RPT)])

        @pl.when(c == 0)
        def _():
            run_direction(pf1l_hbm, dst_hbm, pv1l_hbm, src_hbm, outfl_hbm)
            run_direction(pv2l_hbm, src_hbm, pf2l_hbm, dst_hbm, outvl_hbm)

        @pl.when(c == 1)
        def _():
            run_direction(pf1h_hbm, dst_hbm, pv1h_hbm, src_hbm, outfh_hbm)
            run_direction(pv2h_hbm, src_hbm, pf2h_hbm, dst_hbm, outvh_hbm)

    return edge_kernel(pf1_lo, pf1_hi, pv1_lo, pv1_hi,
                       pv2_lo, pv2_hi, pf2_lo, pf2_hi, dst_p, src_p)


def kernel(variables, factors, edge_index, edge_attr, batch_idx,
           v2f_msg_W, v2f_msg_b, v2f_comb_W, v2f_comb_b,
           f2v_msg_W, f2v_msg_b, f2v_comb_W, f2v_comb_b):
    del edge_attr, batch_idx
    n_vars, _ = variables.shape
    n_facs, _ = factors.shape
    n_edges = edge_index.shape[1]

    ept = _cdiv(n_edges, NS * CH) * CH
    ne_pad = ept * NS
    pad_idx = jnp.full((ne_pad - n_edges,), n_vars, jnp.int32)
    src_p = jnp.concatenate([edge_index[0], pad_idx])
    dst_p = jnp.concatenate([edge_index[1], pad_idx])

    variables_p = jnp.pad(variables, ((0, ROWS_PAD - n_vars), (0, 0)))
    factors_p = jnp.pad(factors, ((0, ROWS_PAD - n_facs), (0, 0)))

    w1a, w1b = v2f_msg_W[:D], v2f_msg_W[D:]
    w2a, w2b = f2v_msg_W[:D], f2v_msg_W[D:]
    b1 = v2f_msg_b.reshape(1, D)
    b2 = f2v_msg_b.reshape(1, D)

    tables = _projections(factors_p, variables_p, w1a, w1b, w2a, w2b, b1, b2)

    afl_p, afh_p, avl_p, avh_p = _sc_edge_aggregate(tables, dst_p, src_p, ept)
    af_lo, af_hi = afl_p[:n_facs], afh_p[:n_facs]
    av_lo, av_hi = avl_p[:n_vars], avh_p[:n_vars]

    wc1a = v2f_comb_W[:D]
    wc1b_lo, wc1b_hi = v2f_comb_W[D:D + H], v2f_comb_W[D + H:]
    wc2a = f2v_comb_W[:D]
    wc2b_lo, wc2b_hi = f2v_comb_W[D:D + H], f2v_comb_W[D + H:]
    b3 = v2f_comb_b.reshape(1, D)
    b4 = f2v_comb_b.reshape(1, D)

    new_factors, new_vars = _combine(factors, af_lo, af_hi, variables,
                                     av_lo, av_hi, wc1a, wc1b_lo, wc1b_hi,
                                     wc2a, wc2b_lo, wc2b_hi, b3, b4)
    return (new_vars, new_factors)
